# Initial kernel scaffold; baseline (speedup 1.0000x reference)
import jax
import jax.numpy as jnp
from jax.experimental import pallas as pl


def kernel(x, W1, bnA_g, bnA_b, prw1, bnB_g, bnB_b, prw2, bnC_g, bnC_b, prw3, prb3, bnD_g, bnD_b, prw4, prb4, bnE_g, bnE_b, Wfc, bfc, Wp1, bp1, Wp2, bp2, Wp3, bp3, W2, bn1_g, bn1_b, bn2_g, bn2_b):
    raise NotImplementedError("write your pallas kernel here")


# speedup vs baseline: 1.6666x; 1.6666x over previous
import functools
import jax
import jax.numpy as jnp
from jax import lax
from jax.experimental import pallas as pl

_M = 10
_B, _C, _H, _W = 4, 96, 224, 224
_HW = _H * _W
_N = _B * _HW
_PB = 1792
_NP = _HW // _PB
_PBH = 6272
_NPH = _HW // _PBH
_CB = 8
_NC = _C // _CB
_EPS = 1e-5

_interp = False


def _p0_body(x_ref, o_ref):
    p = pl.program_id(1)
    s = jnp.sum(x_ref[0], axis=1).reshape(1, 1, _C)

    @pl.when(p == 0)
    def _():
        o_ref[...] = jnp.zeros_like(o_ref)

    o_ref[...] += s


def _p1_body(x_ref, w1_ref, xg_ref, xc1_ref, s_ref, statA_ref, smin_ref,
             smax_ref, ssum_ref, ssq_ref):
    b, p = pl.program_id(0), pl.program_id(1)
    xb = x_ref[0]
    y = lax.dot_general(w1_ref[...], xb, (((1,), (0,)), ((), ())),
                        preferred_element_type=jnp.float32,
                        precision=lax.Precision.HIGHEST)
    xc1_ref[0] = y
    st = jnp.stack([jnp.sum(y, axis=1), jnp.sum(y * y, axis=1)])

    @pl.when((b == 0) & (p == 0))
    def _():
        statA_ref[...] = jnp.zeros_like(statA_ref)

    statA_ref[...] += st

    xg = xg_ref[0, 0]
    na = jnp.sqrt(jnp.sum(xg * xg))
    dot = jnp.sum(xg[:, None] * xb, axis=0)
    nb = jnp.sqrt(jnp.sum(xb * xb, axis=0))
    e8 = 1e-8
    s = dot / (jnp.maximum(na, e8) * jnp.maximum(nb, e8))
    s_ref[0, 0] = s

    @pl.when(p == 0)
    def _():
        smin_ref[...] = jnp.full((1, 1, 1), jnp.inf, jnp.float32)
        smax_ref[...] = jnp.full((1, 1, 1), -jnp.inf, jnp.float32)
        ssum_ref[...] = jnp.zeros((1, 1, 1), jnp.float32)
        ssq_ref[...] = jnp.zeros((1, 1, 1), jnp.float32)

    smin_ref[...] = jnp.minimum(smin_ref[...], jnp.min(s).reshape(1, 1, 1))
    smax_ref[...] = jnp.maximum(smax_ref[...], jnp.max(s).reshape(1, 1, 1))
    ssum_ref[...] += jnp.sum(s).reshape(1, 1, 1)
    ssq_ref[...] += jnp.sum(s * s).reshape(1, 1, 1)


def _conv_body(horiz, dil, has_bias, a_ref, coef_ref, w_ref, bias_ref,
               h_ref, stat_ref):
    b = pl.program_id(1)
    sc = coef_ref[0, 0, :]
    of = coef_ref[0, 1, :]
    a = a_ref[0] * sc[:, None, None] + of[:, None, None]
    pad = 2 * dil
    if horiz:
        z = jnp.zeros((_CB, _H, pad), jnp.float32)
        ap = jnp.concatenate([z, a, z], axis=2)
        h = sum(w_ref[:, t][:, None, None] * ap[:, :, t * dil:t * dil + _W]
                for t in range(5))
    else:
        z = jnp.zeros((_CB, pad, _W), jnp.float32)
        ap = jnp.concatenate([z, a, z], axis=1)
        h = sum(w_ref[:, t][:, None, None] * ap[:, t * dil:t * dil + _H, :]
                for t in range(5))
    if has_bias:
        h = h + bias_ref[0, 0, :][:, None, None]
    h_ref[0] = h
    st = jnp.stack([jnp.sum(h, axis=(1, 2)), jnp.sum(h * h, axis=(1, 2))])

    @pl.when(b == 0)
    def _():
        stat_ref[...] = jnp.zeros_like(stat_ref)

    stat_ref[0] += st


def _hista_body(s_ref, lev_ref, v_ref, vsum_ref):
    p = pl.program_id(1)
    s = s_ref[0, 0]
    L = lev_ref[0, 0]
    diff = jnp.abs(L[None, :] - s[:, None])
    v = jnp.where(diff < 0.5 / _M, 1.0 - diff, 0.0)
    v_ref[0] = v

    @pl.when(p == 0)
    def _():
        vsum_ref[...] = jnp.zeros_like(vsum_ref)

    vsum_ref[0] += jnp.sum(v, axis=0).reshape(1, _M)


def _histb_body(vr_ref, g_ref, rs_ref):
    p = pl.program_id(1)
    u = vr_ref[0]
    g = lax.dot_general(u, u, (((1,), (1,)), ((), ())),
                        preferred_element_type=jnp.float32,
                        precision=lax.Precision.HIGHEST)

    @pl.when(p == 0)
    def _():
        g_ref[...] = jnp.zeros_like(g_ref)
        rs_ref[...] = jnp.zeros_like(rs_ref)

    g_ref[0] += g
    rs_ref[0] += jnp.sum(u, axis=1).reshape(1, _M)


def _fc_body(ch_ref, wfc_ref, bfc_ref, o_ref):
    c0 = ch_ref[:, 0][:, None]
    c1 = ch_ref[:, 1][:, None]
    o_ref[...] = (c0 * wfc_ref[0, :][None, :] + c1 * wfc_ref[1, :][None, :]
                  + bfc_ref[0, :][None, :])


def _small_body(ch_ref, wp1_ref, wp2_ref, wp3_ref, bp_ref, rs_ref, g_ref,
                bn1_ref, lq_ref, b1_ref):
    hp = lax.Precision.HIGHEST
    s1 = jnp.zeros((_C,), jnp.float32)
    ss1 = jnp.zeros((_C,), jnp.float32)
    lps = []
    for b in range(_B):
        chb = ch_ref[b]
        p1 = lax.dot_general(wp1_ref[...], chb, (((1,), (0,)), ((), ())),
                             preferred_element_type=jnp.float32, precision=hp) \
            + bp_ref[0, :][:, None]
        p2 = lax.dot_general(wp2_ref[...], chb, (((1,), (0,)), ((), ())),
                             preferred_element_type=jnp.float32, precision=hp) \
            + bp_ref[1, :][:, None]
        p3 = lax.dot_general(wp3_ref[...], chb, (((1,), (0,)), ((), ())),
                             preferred_element_type=jnp.float32, precision=hp) \
            + bp_ref[2, :][:, None]
        logits = lax.dot_general(p1, p2, (((0,), (0,)), ((), ())),
                                 preferred_element_type=jnp.float32,
                                 precision=hp)
        e = jnp.exp(logits - jnp.max(logits, axis=-1, keepdims=True))
        xm = e / jnp.sum(e, axis=-1, keepdims=True)
        lp = lax.dot_general(p3, xm, (((1,), (0,)), ((), ())),
                             preferred_element_type=jnp.float32, precision=hp)
        lps.append(lp)
        s1 = s1 + jnp.sum(lp * rs_ref[b][None, :], axis=1)
        lg = lax.dot_general(lp, g_ref[b], (((1,), (0,)), ((), ())),
                             preferred_element_type=jnp.float32, precision=hp)
        ss1 = ss1 + jnp.sum(lg * lp, axis=1)
    m1 = s1 / _N
    v1 = ss1 / _N - m1 * m1
    a1 = bn1_ref[0, :] / jnp.sqrt(v1 + _EPS)
    b1_ref[...] = (bn1_ref[1, :] - m1 * a1).reshape(1, _C)
    for b in range(_B):
        lq_ref[b] = lps[b] * a1[:, None]


def _p6_body(x_ref, h4_ref, s_ref, vr_ref, lq_ref, cp_ref, o_ref):
    aE = cp_ref[0, :][:, None]
    bE = cp_ref[1, :][:, None]
    a2w = cp_ref[2, :][:, None]
    b2 = cp_ref[3, :][:, None]
    b1 = cp_ref[4, :][:, None]
    xc = h4_ref[0] * aE + bE
    tex = lax.dot_general(lq_ref[0], vr_ref[0], (((1,), (0,)), ((), ())),
                          preferred_element_type=jnp.float32,
                          precision=lax.Precision.HIGHEST) + b1
    s = s_ref[0, 0][None, :]
    tf = a2w * s + b2
    o_ref[0] = x_ref[0] * jax.nn.sigmoid(tf + tex + xc)


def _coef(stat, g, b):
    m = stat[0] / _N
    v = stat[1] / _N - m * m
    a = g / jnp.sqrt(v + _EPS)
    return jnp.stack([a, b - m * a])


def kernel(x, W1, bnA_g, bnA_b, prw1, bnB_g, bnB_b, prw2, bnC_g, bnC_b, prw3,
           prb3, bnD_g, bnD_b, prw4, prb4, bnE_g, bnE_b, Wfc, bfc, Wp1, bp1,
           Wp2, bp2, Wp3, bp3, W2, bn1_g, bn1_b, bn2_g, bn2_b):
    f32 = jnp.float32
    xf = x.reshape(_B, _C, _HW)

    xgsum = pl.pallas_call(
        _p0_body,
        grid=(_B, _NP),
        in_specs=[pl.BlockSpec((1, _C, _PB), lambda b, p: (b, 0, p))],
        out_specs=pl.BlockSpec((1, 1, _C), lambda b, p: (b, 0, 0)),
        out_shape=jax.ShapeDtypeStruct((_B, 1, _C), f32),
        interpret=_interp,
    )(xf)
    xg = xgsum * (1.0 / _HW)

    xc1, S, statA, smin, smax, ssum, ssq = pl.pallas_call(
        _p1_body,
        grid=(_B, _NP),
        in_specs=[
            pl.BlockSpec((1, _C, _PB), lambda b, p: (b, 0, p)),
            pl.BlockSpec((_C, _C), lambda b, p: (0, 0)),
            pl.BlockSpec((1, 1, _C), lambda b, p: (b, 0, 0)),
        ],
        out_specs=[
            pl.BlockSpec((1, _C, _PB), lambda b, p: (b, 0, p)),
            pl.BlockSpec((1, 1, _PB), lambda b, p: (b, 0, p)),
            pl.BlockSpec((2, _C), lambda b, p: (0, 0)),
            pl.BlockSpec((1, 1, 1), lambda b, p: (b, 0, 0)),
            pl.BlockSpec((1, 1, 1), lambda b, p: (b, 0, 0)),
            pl.BlockSpec((1, 1, 1), lambda b, p: (b, 0, 0)),
            pl.BlockSpec((1, 1, 1), lambda b, p: (b, 0, 0)),
        ],
        out_shape=[
            jax.ShapeDtypeStruct((_B, _C, _HW), f32),
            jax.ShapeDtypeStruct((_B, 1, _HW), f32),
            jax.ShapeDtypeStruct((2, _C), f32),
            jax.ShapeDtypeStruct((_B, 1, 1), f32),
            jax.ShapeDtypeStruct((_B, 1, 1), f32),
            jax.ShapeDtypeStruct((_B, 1, 1), f32),
            jax.ShapeDtypeStruct((_B, 1, 1), f32),
        ],
        interpret=_interp,
    )(xf, W1[:, :, 0, 0], xg)

    def conv_pass(a, coefs, w, bias, horiz, dil):
        body = functools.partial(_conv_body, horiz, dil, bias is not None)
        cf = coefs.reshape(2, _NC, _CB).transpose(1, 0, 2)
        bs = (bias if bias is not None else jnp.zeros((_C,), f32))
        bs = bs.reshape(_NC, 1, _CB)
        h, stat = pl.pallas_call(
            body,
            grid=(_NC, _B),
            in_specs=[
                pl.BlockSpec((1, _CB, _H, _W), lambda c, b: (b, c, 0, 0)),
                pl.BlockSpec((1, 2, _CB), lambda c, b: (c, 0, 0)),
                pl.BlockSpec((_CB, 5), lambda c, b: (c, 0)),
                pl.BlockSpec((1, 1, _CB), lambda c, b: (c, 0, 0)),
            ],
            out_specs=[
                pl.BlockSpec((1, _CB, _H, _W), lambda c, b: (b, c, 0, 0)),
                pl.BlockSpec((1, 2, _CB), lambda c, b: (c, 0, 0)),
            ],
            out_shape=[
                jax.ShapeDtypeStruct((_B, _C, _H, _W), f32),
                jax.ShapeDtypeStruct((_NC, 2, _CB), f32),
            ],
            interpret=_interp,
        )(a.reshape(_B, _C, _H, _W), cf, w, bs)
        return h, stat.transpose(1, 0, 2).reshape(2, _C)

    coefA = _coef(statA, bnA_g, bnA_b)
    h1, statB = conv_pass(xc1, coefA, prw1[:, 0, 0, :], None, True, 1)
    coefB = _coef(statB, bnB_g, bnB_b)
    h2, statC = conv_pass(h1, coefB, prw2[:, 0, :, 0], None, False, 1)
    coefC = _coef(statC, bnC_g, bnC_b)
    h3, statD = conv_pass(h2, coefC, prw3[:, 0, 0, :], prb3, True, 2)
    coefD = _coef(statD, bnD_g, bnD_b)
    h4, statE = conv_pass(h3, coefD, prw4[:, 0, :, 0], prb4, False, 2)
    coefE = _coef(statE, bnE_g, bnE_b)

    mn = smin[:, 0, 0]
    mx = smax[:, 0, 0]
    t = jnp.linspace(0.0, 1.0, _M).astype(f32)
    Level = mn[:, None] + (mx - mn)[:, None] * t[None, :]

    Vflat, Vsum = pl.pallas_call(
        _hista_body,
        grid=(_B, _NPH),
        in_specs=[
            pl.BlockSpec((1, 1, _PBH), lambda b, p: (b, 0, p)),
            pl.BlockSpec((1, 1, _M), lambda b, p: (b, 0, 0)),
        ],
        out_specs=[
            pl.BlockSpec((1, _PBH, _M), lambda b, p: (b, p, 0)),
            pl.BlockSpec((1, 1, _M), lambda b, p: (b, 0, 0)),
        ],
        out_shape=[
            jax.ShapeDtypeStruct((_B, _HW, _M), f32),
            jax.ShapeDtypeStruct((_B, 1, _M), f32),
        ],
        interpret=_interp,
    )(S, Level[:, None, :])

    VR = Vflat.reshape(_B, _M, _HW)

    G, rowsum = pl.pallas_call(
        _histb_body,
        grid=(_B, _NPH),
        in_specs=[pl.BlockSpec((1, _M, _PBH), lambda b, p: (b, 0, p))],
        out_specs=[
            pl.BlockSpec((1, _M, _M), lambda b, p: (b, 0, 0)),
            pl.BlockSpec((1, 1, _M), lambda b, p: (b, 0, 0)),
        ],
        out_shape=[
            jax.ShapeDtypeStruct((_B, _M, _M), f32),
            jax.ShapeDtypeStruct((_B, 1, _M), f32),
        ],
        interpret=_interp,
    )(VR)

    Vsum2 = Vsum[:, 0, :]
    Vtot = jnp.sum(Vsum2, axis=1)
    Chist2 = jnp.stack([Vsum2 / Vtot[:, None], Level], axis=-1).reshape(-1, 2)
    fcout = pl.pallas_call(
        _fc_body,
        in_specs=[
            pl.BlockSpec((_B * _M, 2), lambda: (0, 0)),
            pl.BlockSpec((2, _C), lambda: (0, 0)),
            pl.BlockSpec((1, _C), lambda: (0, 0)),
        ],
        out_specs=pl.BlockSpec((_B * _M, _C), lambda: (0, 0)),
        out_shape=jax.ShapeDtypeStruct((_B * _M, _C), f32),
        interpret=_interp,
    )(Chist2, Wfc.T, bfc.reshape(1, _C))
    Ch = fcout.reshape(_B, _C, _M)

    Lq, b1 = pl.pallas_call(
        _small_body,
        in_specs=[
            pl.BlockSpec((_B, _C, _M), lambda: (0, 0, 0)),
            pl.BlockSpec((_C, _C), lambda: (0, 0)),
            pl.BlockSpec((_C, _C), lambda: (0, 0)),
            pl.BlockSpec((_C, _C), lambda: (0, 0)),
            pl.BlockSpec((3, _C), lambda: (0, 0)),
            pl.BlockSpec((_B, _M), lambda: (0, 0)),
            pl.BlockSpec((_B, _M, _M), lambda: (0, 0, 0)),
            pl.BlockSpec((2, _C), lambda: (0, 0)),
        ],
        out_specs=[
            pl.BlockSpec((_B, _C, _M), lambda: (0, 0, 0)),
            pl.BlockSpec((1, _C), lambda: (0, 0)),
        ],
        out_shape=[
            jax.ShapeDtypeStruct((_B, _C, _M), f32),
            jax.ShapeDtypeStruct((1, _C), f32),
        ],
        interpret=_interp,
    )(Ch, Wp1, Wp2, Wp3, jnp.stack([bp1, bp2, bp3]), rowsum[:, 0, :], G,
      jnp.stack([bn1_g, bn1_b]))

    w2 = W2[:, 0, 0, 0]
    sS = jnp.sum(ssum[:, 0, 0])
    ssS = jnp.sum(ssq[:, 0, 0])
    mS = sS / _N
    vS = ssS / _N - mS * mS
    m2 = w2 * mS
    v2 = w2 * w2 * vS
    a2 = bn2_g / jnp.sqrt(v2 + _EPS)
    cpack = jnp.stack([coefE[0], coefE[1], a2 * w2, bn2_b - m2 * a2, b1[0],
                       jnp.zeros_like(w2)])

    out = pl.pallas_call(
        _p6_body,
        grid=(_B, _NP),
        in_specs=[
            pl.BlockSpec((1, _C, _PB), lambda b, p: (b, 0, p)),
            pl.BlockSpec((1, _C, _PB), lambda b, p: (b, 0, p)),
            pl.BlockSpec((1, 1, _PB), lambda b, p: (b, 0, p)),
            pl.BlockSpec((1, _M, _PB), lambda b, p: (b, 0, p)),
            pl.BlockSpec((1, _C, _M), lambda b, p: (b, 0, 0)),
            pl.BlockSpec((6, _C), lambda b, p: (0, 0)),
        ],
        out_specs=pl.BlockSpec((1, _C, _PB), lambda b, p: (b, 0, p)),
        out_shape=jax.ShapeDtypeStruct((_B, _C, _HW), f32),
        interpret=_interp,
    )(xf, h4.reshape(_B, _C, _HW), S, VR, Lq, cpack)

    return out.reshape(_B, _C, _H, _W)


# speedup vs baseline: 1.7149x; 1.0289x over previous
import functools
import jax
import jax.numpy as jnp
from jax import lax
from jax.experimental import pallas as pl

_M = 10
_B, _C, _H, _W = 4, 96, 224, 224
_HW = _H * _W
_N = _B * _HW
_PB = 1792
_NP = _HW // _PB
_PBH = 6272
_NPH = _HW // _PBH
_CB = 8
_NC = _C // _CB
_EPS = 1e-5

_interp = False
_BF = jnp.bfloat16


def _p1_body(x_ref, w1_ref, xc1_ref, nb2_ref, statA_ref, xg_ref):
    b, p = pl.program_id(0), pl.program_id(1)
    del b
    xb = x_ref[0]
    y = lax.dot_general(w1_ref[...], xb, (((1,), (0,)), ((), ())),
                        preferred_element_type=jnp.float32,
                        precision=lax.Precision.HIGHEST)
    xc1_ref[0] = y.astype(_BF)
    nb2_ref[0, 0] = jnp.sum(xb * xb, axis=0)
    st = jnp.stack([jnp.sum(y, axis=1), jnp.sum(y * y, axis=1)])

    @pl.when(p == 0)
    def _():
        statA_ref[...] = jnp.zeros_like(statA_ref)
        xg_ref[...] = jnp.zeros_like(xg_ref)

    statA_ref[0] += st
    xg_ref[...] += jnp.sum(xb, axis=1).reshape(1, 1, _C)


def _phs_body(x_ref, xg_ref, nb2_ref, s_ref, smin_ref, smax_ref, ssum_ref,
              ssq_ref):
    p = pl.program_id(1)
    xb = x_ref[0]
    xg = xg_ref[0, 0]
    na = jnp.sqrt(jnp.sum(xg * xg))
    dot = jnp.sum(xg[:, None] * xb, axis=0)
    nb = jnp.sqrt(nb2_ref[0, 0])
    e8 = 1e-8
    s = dot / (jnp.maximum(na, e8) * jnp.maximum(nb, e8))
    s_ref[0, 0] = s

    @pl.when(p == 0)
    def _():
        smin_ref[...] = jnp.full((1, 1, 1), jnp.inf, jnp.float32)
        smax_ref[...] = jnp.full((1, 1, 1), -jnp.inf, jnp.float32)
        ssum_ref[...] = jnp.zeros((1, 1, 1), jnp.float32)
        ssq_ref[...] = jnp.zeros((1, 1, 1), jnp.float32)

    smin_ref[...] = jnp.minimum(smin_ref[...], jnp.min(s).reshape(1, 1, 1))
    smax_ref[...] = jnp.maximum(smax_ref[...], jnp.max(s).reshape(1, 1, 1))
    ssum_ref[...] += jnp.sum(s).reshape(1, 1, 1)
    ssq_ref[...] += jnp.sum(s * s).reshape(1, 1, 1)


def _conv_body(horiz, dil, has_bias, a_ref, coef_ref, w_ref, bias_ref,
               h_ref, stat_ref):
    b = pl.program_id(1)
    sc = coef_ref[0, 0, :]
    of = coef_ref[0, 1, :]
    a = a_ref[0].astype(jnp.float32) * sc[:, None, None] + of[:, None, None]
    pad = 2 * dil
    if horiz:
        z = jnp.zeros((_CB, _H, pad), jnp.float32)
        ap = jnp.concatenate([z, a, z], axis=2)
        h = sum(w_ref[:, t][:, None, None] * ap[:, :, t * dil:t * dil + _W]
                for t in range(5))
    else:
        z = jnp.zeros((_CB, pad, _W), jnp.float32)
        ap = jnp.concatenate([z, a, z], axis=1)
        h = sum(w_ref[:, t][:, None, None] * ap[:, t * dil:t * dil + _H, :]
                for t in range(5))
    if has_bias:
        h = h + bias_ref[0, 0, :][:, None, None]
    h_ref[0] = h.astype(_BF)
    st = jnp.stack([jnp.sum(h, axis=(1, 2)), jnp.sum(h * h, axis=(1, 2))])

    @pl.when(b == 0)
    def _():
        stat_ref[...] = jnp.zeros_like(stat_ref)

    stat_ref[0] += st


def _hista_body(s_ref, lev_ref, v_ref, vsum_ref):
    p = pl.program_id(1)
    s = s_ref[0, 0]
    L = lev_ref[0, 0]
    diff = jnp.abs(L[None, :] - s[:, None])
    v = jnp.where(diff < 0.5 / _M, 1.0 - diff, 0.0)
    v_ref[0] = v.astype(_BF)

    @pl.when(p == 0)
    def _():
        vsum_ref[...] = jnp.zeros_like(vsum_ref)

    vsum_ref[0] += jnp.sum(v, axis=0).reshape(1, _M)


def _histb_body(vr_ref, g_ref, rs_ref):
    p = pl.program_id(1)
    u = vr_ref[0]
    g = lax.dot_general(u, u, (((1,), (1,)), ((), ())),
                        preferred_element_type=jnp.float32)
    uf = u.astype(jnp.float32)

    @pl.when(p == 0)
    def _():
        g_ref[...] = jnp.zeros_like(g_ref)
        rs_ref[...] = jnp.zeros_like(rs_ref)

    g_ref[0] += g
    rs_ref[0] += jnp.sum(uf, axis=1).reshape(1, _M)


def _fc_body(ch_ref, wfc_ref, bfc_ref, o_ref):
    c0 = ch_ref[:, 0][:, None]
    c1 = ch_ref[:, 1][:, None]
    o_ref[...] = (c0 * wfc_ref[0, :][None, :] + c1 * wfc_ref[1, :][None, :]
                  + bfc_ref[0, :][None, :])


def _small_body(ch_ref, wp1_ref, wp2_ref, wp3_ref, bp_ref, rs_ref, g_ref,
                bn1_ref, lq_ref, b1_ref):
    hp = lax.Precision.HIGHEST
    s1 = jnp.zeros((_C,), jnp.float32)
    ss1 = jnp.zeros((_C,), jnp.float32)
    lps = []
    for b in range(_B):
        chb = ch_ref[b]
        p1 = lax.dot_general(wp1_ref[...], chb, (((1,), (0,)), ((), ())),
                             preferred_element_type=jnp.float32, precision=hp) \
            + bp_ref[0, :][:, None]
        p2 = lax.dot_general(wp2_ref[...], chb, (((1,), (0,)), ((), ())),
                             preferred_element_type=jnp.float32, precision=hp) \
            + bp_ref[1, :][:, None]
        p3 = lax.dot_general(wp3_ref[...], chb, (((1,), (0,)), ((), ())),
                             preferred_element_type=jnp.float32, precision=hp) \
            + bp_ref[2, :][:, None]
        logits = lax.dot_general(p1, p2, (((0,), (0,)), ((), ())),
                                 preferred_element_type=jnp.float32,
                                 precision=hp)
        e = jnp.exp(logits - jnp.max(logits, axis=-1, keepdims=True))
        xm = e / jnp.sum(e, axis=-1, keepdims=True)
        lp = lax.dot_general(p3, xm, (((1,), (0,)), ((), ())),
                             preferred_element_type=jnp.float32, precision=hp)
        lps.append(lp)
        s1 = s1 + jnp.sum(lp * rs_ref[b][None, :], axis=1)
        lg = lax.dot_general(lp, g_ref[b], (((1,), (0,)), ((), ())),
                             preferred_element_type=jnp.float32, precision=hp)
        ss1 = ss1 + jnp.sum(lg * lp, axis=1)
    m1 = s1 / _N
    v1 = ss1 / _N - m1 * m1
    a1 = bn1_ref[0, :] / jnp.sqrt(v1 + _EPS)
    b1_ref[...] = (bn1_ref[1, :] - m1 * a1).reshape(1, _C)
    for b in range(_B):
        lq_ref[b] = lps[b] * a1[:, None]


def _p6_body(x_ref, h4_ref, s_ref, vr_ref, lq_ref, cp_ref, o_ref):
    aE = cp_ref[0, :][:, None]
    bE = cp_ref[1, :][:, None]
    a2w = cp_ref[2, :][:, None]
    b2 = cp_ref[3, :][:, None]
    b1 = cp_ref[4, :][:, None]
    xc = h4_ref[0].astype(jnp.float32) * aE + bE
    tex = lax.dot_general(lq_ref[0].astype(_BF), vr_ref[0],
                          (((1,), (0,)), ((), ())),
                          preferred_element_type=jnp.float32) + b1
    s = s_ref[0, 0][None, :]
    tf = a2w * s + b2
    o_ref[0] = x_ref[0] * jax.nn.sigmoid(tf + tex + xc)


def _coef(stat, g, b):
    m = stat[0] / _N
    v = stat[1] / _N - m * m
    a = g / jnp.sqrt(v + _EPS)
    return jnp.stack([a, b - m * a])


def kernel(x, W1, bnA_g, bnA_b, prw1, bnB_g, bnB_b, prw2, bnC_g, bnC_b, prw3,
           prb3, bnD_g, bnD_b, prw4, prb4, bnE_g, bnE_b, Wfc, bfc, Wp1, bp1,
           Wp2, bp2, Wp3, bp3, W2, bn1_g, bn1_b, bn2_g, bn2_b):
    f32 = jnp.float32
    xf = x.reshape(_B, _C, _HW)

    xc1, nb2, statA, xgsum = pl.pallas_call(
        _p1_body,
        grid=(_B, _NP),
        in_specs=[
            pl.BlockSpec((1, _C, _PB), lambda b, p: (b, 0, p)),
            pl.BlockSpec((_C, _C), lambda b, p: (0, 0)),
        ],
        out_specs=[
            pl.BlockSpec((1, _C, _PB), lambda b, p: (b, 0, p)),
            pl.BlockSpec((1, 1, _PB), lambda b, p: (b, 0, p)),
            pl.BlockSpec((1, 2, _C), lambda b, p: (0, 0, 0)),
            pl.BlockSpec((1, 1, _C), lambda b, p: (b, 0, 0)),
        ],
        out_shape=[
            jax.ShapeDtypeStruct((_B, _C, _HW), _BF),
            jax.ShapeDtypeStruct((_B, 1, _HW), f32),
            jax.ShapeDtypeStruct((1, 2, _C), f32),
            jax.ShapeDtypeStruct((_B, 1, _C), f32),
        ],
        interpret=_interp,
    )(xf, W1[:, :, 0, 0])
    xg = xgsum * (1.0 / _HW)

    S, smin, smax, ssum, ssq = pl.pallas_call(
        _phs_body,
        grid=(_B, _NP),
        in_specs=[
            pl.BlockSpec((1, _C, _PB), lambda b, p: (b, 0, p)),
            pl.BlockSpec((1, 1, _C), lambda b, p: (b, 0, 0)),
            pl.BlockSpec((1, 1, _PB), lambda b, p: (b, 0, p)),
        ],
        out_specs=[
            pl.BlockSpec((1, 1, _PB), lambda b, p: (b, 0, p)),
            pl.BlockSpec((1, 1, 1), lambda b, p: (b, 0, 0)),
            pl.BlockSpec((1, 1, 1), lambda b, p: (b, 0, 0)),
            pl.BlockSpec((1, 1, 1), lambda b, p: (b, 0, 0)),
            pl.BlockSpec((1, 1, 1), lambda b, p: (b, 0, 0)),
        ],
        out_shape=[
            jax.ShapeDtypeStruct((_B, 1, _HW), f32),
            jax.ShapeDtypeStruct((_B, 1, 1), f32),
            jax.ShapeDtypeStruct((_B, 1, 1), f32),
            jax.ShapeDtypeStruct((_B, 1, 1), f32),
            jax.ShapeDtypeStruct((_B, 1, 1), f32),
        ],
        interpret=_interp,
    )(xf, xg, nb2)

    def conv_pass(a, coefs, w, bias, horiz, dil):
        body = functools.partial(_conv_body, horiz, dil, bias is not None)
        cf = coefs.reshape(2, _NC, _CB).transpose(1, 0, 2)
        bs = (bias if bias is not None else jnp.zeros((_C,), f32))
        bs = bs.reshape(_NC, 1, _CB)
        h, stat = pl.pallas_call(
            body,
            grid=(_NC, _B),
            in_specs=[
                pl.BlockSpec((1, _CB, _H, _W), lambda c, b: (b, c, 0, 0)),
                pl.BlockSpec((1, 2, _CB), lambda c, b: (c, 0, 0)),
                pl.BlockSpec((_CB, 5), lambda c, b: (c, 0)),
                pl.BlockSpec((1, 1, _CB), lambda c, b: (c, 0, 0)),
            ],
            out_specs=[
                pl.BlockSpec((1, _CB, _H, _W), lambda c, b: (b, c, 0, 0)),
                pl.BlockSpec((1, 2, _CB), lambda c, b: (c, 0, 0)),
            ],
            out_shape=[
                jax.ShapeDtypeStruct((_B, _C, _H, _W), _BF),
                jax.ShapeDtypeStruct((_NC, 2, _CB), f32),
            ],
            interpret=_interp,
        )(a.reshape(_B, _C, _H, _W), cf, w, bs)
        return h, stat.transpose(1, 0, 2).reshape(2, _C)

    coefA = _coef(statA[0], bnA_g, bnA_b)
    h1, statB = conv_pass(xc1, coefA, prw1[:, 0, 0, :], None, True, 1)
    coefB = _coef(statB, bnB_g, bnB_b)
    h2, statC = conv_pass(h1, coefB, prw2[:, 0, :, 0], None, False, 1)
    coefC = _coef(statC, bnC_g, bnC_b)
    h3, statD = conv_pass(h2, coefC, prw3[:, 0, 0, :], prb3, True, 2)
    coefD = _coef(statD, bnD_g, bnD_b)
    h4, statE = conv_pass(h3, coefD, prw4[:, 0, :, 0], prb4, False, 2)
    coefE = _coef(statE, bnE_g, bnE_b)

    mn = smin[:, 0, 0]
    mx = smax[:, 0, 0]
    t = jnp.linspace(0.0, 1.0, _M).astype(f32)
    Level = mn[:, None] + (mx - mn)[:, None] * t[None, :]

    Vflat, Vsum = pl.pallas_call(
        _hista_body,
        grid=(_B, _NPH),
        in_specs=[
            pl.BlockSpec((1, 1, _PBH), lambda b, p: (b, 0, p)),
            pl.BlockSpec((1, 1, _M), lambda b, p: (b, 0, 0)),
        ],
        out_specs=[
            pl.BlockSpec((1, _PBH, _M), lambda b, p: (b, p, 0)),
            pl.BlockSpec((1, 1, _M), lambda b, p: (b, 0, 0)),
        ],
        out_shape=[
            jax.ShapeDtypeStruct((_B, _HW, _M), _BF),
            jax.ShapeDtypeStruct((_B, 1, _M), f32),
        ],
        interpret=_interp,
    )(S, Level[:, None, :])

    VR = Vflat.reshape(_B, _M, _HW)

    G, rowsum = pl.pallas_call(
        _histb_body,
        grid=(_B, _NPH),
        in_specs=[pl.BlockSpec((1, _M, _PBH), lambda b, p: (b, 0, p))],
        out_specs=[
            pl.BlockSpec((1, _M, _M), lambda b, p: (b, 0, 0)),
            pl.BlockSpec((1, 1, _M), lambda b, p: (b, 0, 0)),
        ],
        out_shape=[
            jax.ShapeDtypeStruct((_B, _M, _M), f32),
            jax.ShapeDtypeStruct((_B, 1, _M), f32),
        ],
        interpret=_interp,
    )(VR)

    Vsum2 = Vsum[:, 0, :]
    Vtot = jnp.sum(Vsum2, axis=1)
    Chist2 = jnp.stack([Vsum2 / Vtot[:, None], Level], axis=-1).reshape(-1, 2)
    fcout = pl.pallas_call(
        _fc_body,
        in_specs=[
            pl.BlockSpec((_B * _M, 2), lambda: (0, 0)),
            pl.BlockSpec((2, _C), lambda: (0, 0)),
            pl.BlockSpec((1, _C), lambda: (0, 0)),
        ],
        out_specs=pl.BlockSpec((_B * _M, _C), lambda: (0, 0)),
        out_shape=jax.ShapeDtypeStruct((_B * _M, _C), f32),
        interpret=_interp,
    )(Chist2, Wfc.T, bfc.reshape(1, _C))
    Ch = fcout.reshape(_B, _C, _M)

    Lq, b1 = pl.pallas_call(
        _small_body,
        in_specs=[
            pl.BlockSpec((_B, _C, _M), lambda: (0, 0, 0)),
            pl.BlockSpec((_C, _C), lambda: (0, 0)),
            pl.BlockSpec((_C, _C), lambda: (0, 0)),
            pl.BlockSpec((_C, _C), lambda: (0, 0)),
            pl.BlockSpec((3, _C), lambda: (0, 0)),
            pl.BlockSpec((_B, _M), lambda: (0, 0)),
            pl.BlockSpec((_B, _M, _M), lambda: (0, 0, 0)),
            pl.BlockSpec((2, _C), lambda: (0, 0)),
        ],
        out_specs=[
            pl.BlockSpec((_B, _C, _M), lambda: (0, 0, 0)),
            pl.BlockSpec((1, _C), lambda: (0, 0)),
        ],
        out_shape=[
            jax.ShapeDtypeStruct((_B, _C, _M), f32),
            jax.ShapeDtypeStruct((1, _C), f32),
        ],
        interpret=_interp,
    )(Ch, Wp1, Wp2, Wp3, jnp.stack([bp1, bp2, bp3]), rowsum[:, 0, :], G,
      jnp.stack([bn1_g, bn1_b]))

    w2 = W2[:, 0, 0, 0]
    sS = jnp.sum(ssum[:, 0, 0])
    ssS = jnp.sum(ssq[:, 0, 0])
    mS = sS / _N
    vS = ssS / _N - mS * mS
    m2 = w2 * mS
    v2 = w2 * w2 * vS
    a2 = bn2_g / jnp.sqrt(v2 + _EPS)
    cpack = jnp.stack([coefE[0], coefE[1], a2 * w2, bn2_b - m2 * a2, b1[0],
                       jnp.zeros_like(w2)])

    out = pl.pallas_call(
        _p6_body,
        grid=(_B, _NP),
        in_specs=[
            pl.BlockSpec((1, _C, _PB), lambda b, p: (b, 0, p)),
            pl.BlockSpec((1, _C, _PB), lambda b, p: (b, 0, p)),
            pl.BlockSpec((1, 1, _PB), lambda b, p: (b, 0, p)),
            pl.BlockSpec((1, _M, _PB), lambda b, p: (b, 0, p)),
            pl.BlockSpec((1, _C, _M), lambda b, p: (b, 0, 0)),
            pl.BlockSpec((6, _C), lambda b, p: (0, 0)),
        ],
        out_specs=pl.BlockSpec((1, _C, _PB), lambda b, p: (b, 0, p)),
        out_shape=jax.ShapeDtypeStruct((_B, _C, _HW), f32),
        interpret=_interp,
    )(xf, h4.reshape(_B, _C, _HW), S, VR, Lq, cpack)

    return out.reshape(_B, _C, _H, _W)


# speedup vs baseline: 2.0717x; 1.2081x over previous
import functools
import jax
import jax.numpy as jnp
from jax import lax
from jax.experimental import pallas as pl

_M = 10
_B, _C, _H, _W = 4, 96, 224, 224
_HW = _H * _W
_N = _B * _HW
_PB = 1792
_NP = _HW // _PB
_PBH = 6272
_NPH = _HW // _PBH
_CB = 8
_NC = _C // _CB
_EPS = 1e-5

_interp = False
_BF = jnp.bfloat16


def _p1_body(x_ref, w1_ref, xc1_ref, nb2_ref, statA_ref, xg_ref):
    b, p = pl.program_id(0), pl.program_id(1)
    del b
    xb = x_ref[0]
    y = lax.dot_general(w1_ref[...], xb, (((1,), (0,)), ((), ())),
                        preferred_element_type=jnp.float32)
    xc1_ref[0] = y.astype(_BF)
    nb2_ref[0, 0] = jnp.sum(xb * xb, axis=0)
    st = jnp.stack([jnp.sum(y, axis=1), jnp.sum(y * y, axis=1)])

    @pl.when(p == 0)
    def _():
        statA_ref[...] = jnp.zeros_like(statA_ref)
        xg_ref[...] = jnp.zeros_like(xg_ref)

    statA_ref[0] += st
    xg_ref[...] += jnp.sum(xb, axis=1).reshape(1, 1, _C)


def _phs_body(x_ref, xg_ref, nb2_ref, s_ref, smin_ref, smax_ref, ssum_ref,
              ssq_ref):
    p = pl.program_id(1)
    xb = x_ref[0]
    xg = xg_ref[0, 0]
    na = jnp.sqrt(jnp.sum(xg * xg))
    dot = jnp.sum(xg[:, None] * xb, axis=0)
    nb = jnp.sqrt(nb2_ref[0, 0])
    e8 = 1e-8
    s = dot / (jnp.maximum(na, e8) * jnp.maximum(nb, e8))
    s_ref[0, 0] = s

    @pl.when(p == 0)
    def _():
        smin_ref[...] = jnp.full((1, 1, 1), jnp.inf, jnp.float32)
        smax_ref[...] = jnp.full((1, 1, 1), -jnp.inf, jnp.float32)
        ssum_ref[...] = jnp.zeros((1, 1, 1), jnp.float32)
        ssq_ref[...] = jnp.zeros((1, 1, 1), jnp.float32)

    smin_ref[...] = jnp.minimum(smin_ref[...], jnp.min(s).reshape(1, 1, 1))
    smax_ref[...] = jnp.maximum(smax_ref[...], jnp.max(s).reshape(1, 1, 1))
    ssum_ref[...] += jnp.sum(s).reshape(1, 1, 1)
    ssq_ref[...] += jnp.sum(s * s).reshape(1, 1, 1)


def _conv_body(horiz, dil, has_bias, a_ref, coef_ref, w_ref, bias_ref,
               h_ref, stat_ref):
    b = pl.program_id(1)
    sc = coef_ref[0, 0, :].astype(_BF)
    of = coef_ref[0, 1, :].astype(_BF)
    wb = w_ref[...].astype(_BF)
    a = a_ref[0] * sc[:, None, None] + of[:, None, None]
    pad = 2 * dil
    if horiz:
        z = jnp.zeros((_CB, _H, pad), _BF)
        ap = jnp.concatenate([z, a, z], axis=2)
        h = sum(wb[:, t][:, None, None] * ap[:, :, t * dil:t * dil + _W]
                for t in range(5))
    else:
        z = jnp.zeros((_CB, pad, _W), _BF)
        ap = jnp.concatenate([z, a, z], axis=1)
        h = sum(wb[:, t][:, None, None] * ap[:, t * dil:t * dil + _H, :]
                for t in range(5))
    if has_bias:
        h = h + bias_ref[0, 0, :].astype(_BF)[:, None, None]
    h_ref[0] = h
    hf = h.astype(jnp.float32)
    st = jnp.stack([jnp.sum(hf, axis=(1, 2)), jnp.sum(hf * hf, axis=(1, 2))])

    @pl.when(b == 0)
    def _():
        stat_ref[...] = jnp.zeros_like(stat_ref)

    stat_ref[0] += st


def _hista_body(s_ref, lev_ref, v_ref, vsum_ref):
    p = pl.program_id(1)
    s = s_ref[0, 0]
    L = lev_ref[0, 0]
    diff = jnp.abs(L[None, :] - s[:, None])
    v = jnp.where(diff < 0.5 / _M, 1.0 - diff, 0.0)
    v_ref[0] = v.astype(_BF)

    @pl.when(p == 0)
    def _():
        vsum_ref[...] = jnp.zeros_like(vsum_ref)

    vsum_ref[0] += jnp.sum(v, axis=0).reshape(1, _M)


def _histb_body(vr_ref, g_ref, rs_ref):
    p = pl.program_id(1)
    u = vr_ref[0]
    g = lax.dot_general(u, u, (((1,), (1,)), ((), ())),
                        preferred_element_type=jnp.float32)
    uf = u.astype(jnp.float32)

    @pl.when(p == 0)
    def _():
        g_ref[...] = jnp.zeros_like(g_ref)
        rs_ref[...] = jnp.zeros_like(rs_ref)

    g_ref[0] += g
    rs_ref[0] += jnp.sum(uf, axis=1).reshape(1, _M)


def _fc_body(ch_ref, wfc_ref, bfc_ref, o_ref):
    c0 = ch_ref[:, 0][:, None]
    c1 = ch_ref[:, 1][:, None]
    o_ref[...] = (c0 * wfc_ref[0, :][None, :] + c1 * wfc_ref[1, :][None, :]
                  + bfc_ref[0, :][None, :])


def _small_body(ch_ref, wp1_ref, wp2_ref, wp3_ref, bp_ref, rs_ref, g_ref,
                bn1_ref, lq_ref, b1_ref):
    hp = lax.Precision.HIGHEST
    s1 = jnp.zeros((_C,), jnp.float32)
    ss1 = jnp.zeros((_C,), jnp.float32)
    lps = []
    for b in range(_B):
        chb = ch_ref[b]
        p1 = lax.dot_general(wp1_ref[...], chb, (((1,), (0,)), ((), ())),
                             preferred_element_type=jnp.float32, precision=hp) \
            + bp_ref[0, :][:, None]
        p2 = lax.dot_general(wp2_ref[...], chb, (((1,), (0,)), ((), ())),
                             preferred_element_type=jnp.float32, precision=hp) \
            + bp_ref[1, :][:, None]
        p3 = lax.dot_general(wp3_ref[...], chb, (((1,), (0,)), ((), ())),
                             preferred_element_type=jnp.float32, precision=hp) \
            + bp_ref[2, :][:, None]
        logits = lax.dot_general(p1, p2, (((0,), (0,)), ((), ())),
                                 preferred_element_type=jnp.float32,
                                 precision=hp)
        e = jnp.exp(logits - jnp.max(logits, axis=-1, keepdims=True))
        xm = e / jnp.sum(e, axis=-1, keepdims=True)
        lp = lax.dot_general(p3, xm, (((1,), (0,)), ((), ())),
                             preferred_element_type=jnp.float32, precision=hp)
        lps.append(lp)
        s1 = s1 + jnp.sum(lp * rs_ref[b][None, :], axis=1)
        lg = lax.dot_general(lp, g_ref[b], (((1,), (0,)), ((), ())),
                             preferred_element_type=jnp.float32, precision=hp)
        ss1 = ss1 + jnp.sum(lg * lp, axis=1)
    m1 = s1 / _N
    v1 = ss1 / _N - m1 * m1
    a1 = bn1_ref[0, :] / jnp.sqrt(v1 + _EPS)
    b1_ref[...] = (bn1_ref[1, :] - m1 * a1).reshape(1, _C)
    for b in range(_B):
        lq_ref[b] = lps[b] * a1[:, None]


def _p6_body(x_ref, h4_ref, s_ref, vr_ref, lq_ref, cp_ref, o_ref):
    aE = cp_ref[0, :][:, None]
    bE = cp_ref[1, :][:, None]
    a2w = cp_ref[2, :][:, None]
    b2 = cp_ref[3, :][:, None]
    b1 = cp_ref[4, :][:, None]
    xc = h4_ref[0].astype(jnp.float32) * aE + bE
    tex = lax.dot_general(lq_ref[0].astype(_BF), vr_ref[0],
                          (((1,), (0,)), ((), ())),
                          preferred_element_type=jnp.float32) + b1
    s = s_ref[0, 0][None, :]
    tf = a2w * s + b2
    o_ref[0] = x_ref[0] * jax.nn.sigmoid(tf + tex + xc)


def _coef(stat, g, b):
    m = stat[0] / _N
    v = stat[1] / _N - m * m
    a = g / jnp.sqrt(v + _EPS)
    return jnp.stack([a, b - m * a])


def kernel(x, W1, bnA_g, bnA_b, prw1, bnB_g, bnB_b, prw2, bnC_g, bnC_b, prw3,
           prb3, bnD_g, bnD_b, prw4, prb4, bnE_g, bnE_b, Wfc, bfc, Wp1, bp1,
           Wp2, bp2, Wp3, bp3, W2, bn1_g, bn1_b, bn2_g, bn2_b):
    f32 = jnp.float32
    xf = x.reshape(_B, _C, _HW)

    xc1, nb2, statA, xgsum = pl.pallas_call(
        _p1_body,
        grid=(_B, _NP),
        in_specs=[
            pl.BlockSpec((1, _C, _PB), lambda b, p: (b, 0, p)),
            pl.BlockSpec((_C, _C), lambda b, p: (0, 0)),
        ],
        out_specs=[
            pl.BlockSpec((1, _C, _PB), lambda b, p: (b, 0, p)),
            pl.BlockSpec((1, 1, _PB), lambda b, p: (b, 0, p)),
            pl.BlockSpec((1, 2, _C), lambda b, p: (0, 0, 0)),
            pl.BlockSpec((1, 1, _C), lambda b, p: (b, 0, 0)),
        ],
        out_shape=[
            jax.ShapeDtypeStruct((_B, _C, _HW), _BF),
            jax.ShapeDtypeStruct((_B, 1, _HW), f32),
            jax.ShapeDtypeStruct((1, 2, _C), f32),
            jax.ShapeDtypeStruct((_B, 1, _C), f32),
        ],
        interpret=_interp,
    )(xf, W1[:, :, 0, 0])
    xg = xgsum * (1.0 / _HW)

    S, smin, smax, ssum, ssq = pl.pallas_call(
        _phs_body,
        grid=(_B, _NP),
        in_specs=[
            pl.BlockSpec((1, _C, _PB), lambda b, p: (b, 0, p)),
            pl.BlockSpec((1, 1, _C), lambda b, p: (b, 0, 0)),
            pl.BlockSpec((1, 1, _PB), lambda b, p: (b, 0, p)),
        ],
        out_specs=[
            pl.BlockSpec((1, 1, _PB), lambda b, p: (b, 0, p)),
            pl.BlockSpec((1, 1, 1), lambda b, p: (b, 0, 0)),
            pl.BlockSpec((1, 1, 1), lambda b, p: (b, 0, 0)),
            pl.BlockSpec((1, 1, 1), lambda b, p: (b, 0, 0)),
            pl.BlockSpec((1, 1, 1), lambda b, p: (b, 0, 0)),
        ],
        out_shape=[
            jax.ShapeDtypeStruct((_B, 1, _HW), f32),
            jax.ShapeDtypeStruct((_B, 1, 1), f32),
            jax.ShapeDtypeStruct((_B, 1, 1), f32),
            jax.ShapeDtypeStruct((_B, 1, 1), f32),
            jax.ShapeDtypeStruct((_B, 1, 1), f32),
        ],
        interpret=_interp,
    )(xf, xg, nb2)

    def conv_pass(a, coefs, w, bias, horiz, dil):
        body = functools.partial(_conv_body, horiz, dil, bias is not None)
        cf = coefs.reshape(2, _NC, _CB).transpose(1, 0, 2)
        bs = (bias if bias is not None else jnp.zeros((_C,), f32))
        bs = bs.reshape(_NC, 1, _CB)
        h, stat = pl.pallas_call(
            body,
            grid=(_NC, _B),
            in_specs=[
                pl.BlockSpec((1, _CB, _H, _W), lambda c, b: (b, c, 0, 0)),
                pl.BlockSpec((1, 2, _CB), lambda c, b: (c, 0, 0)),
                pl.BlockSpec((_CB, 5), lambda c, b: (c, 0)),
                pl.BlockSpec((1, 1, _CB), lambda c, b: (c, 0, 0)),
            ],
            out_specs=[
                pl.BlockSpec((1, _CB, _H, _W), lambda c, b: (b, c, 0, 0)),
                pl.BlockSpec((1, 2, _CB), lambda c, b: (c, 0, 0)),
            ],
            out_shape=[
                jax.ShapeDtypeStruct((_B, _C, _H, _W), _BF),
                jax.ShapeDtypeStruct((_NC, 2, _CB), f32),
            ],
            interpret=_interp,
        )(a.reshape(_B, _C, _H, _W), cf, w, bs)
        return h, stat.transpose(1, 0, 2).reshape(2, _C)

    coefA = _coef(statA[0], bnA_g, bnA_b)
    h1, statB = conv_pass(xc1, coefA, prw1[:, 0, 0, :], None, True, 1)
    coefB = _coef(statB, bnB_g, bnB_b)
    h2, statC = conv_pass(h1, coefB, prw2[:, 0, :, 0], None, False, 1)
    coefC = _coef(statC, bnC_g, bnC_b)
    h3, statD = conv_pass(h2, coefC, prw3[:, 0, 0, :], prb3, True, 2)
    coefD = _coef(statD, bnD_g, bnD_b)
    h4, statE = conv_pass(h3, coefD, prw4[:, 0, :, 0], prb4, False, 2)
    coefE = _coef(statE, bnE_g, bnE_b)

    mn = smin[:, 0, 0]
    mx = smax[:, 0, 0]
    t = jnp.linspace(0.0, 1.0, _M).astype(f32)
    Level = mn[:, None] + (mx - mn)[:, None] * t[None, :]

    Vflat, Vsum = pl.pallas_call(
        _hista_body,
        grid=(_B, _NPH),
        in_specs=[
            pl.BlockSpec((1, 1, _PBH), lambda b, p: (b, 0, p)),
            pl.BlockSpec((1, 1, _M), lambda b, p: (b, 0, 0)),
        ],
        out_specs=[
            pl.BlockSpec((1, _PBH, _M), lambda b, p: (b, p, 0)),
            pl.BlockSpec((1, 1, _M), lambda b, p: (b, 0, 0)),
        ],
        out_shape=[
            jax.ShapeDtypeStruct((_B, _HW, _M), _BF),
            jax.ShapeDtypeStruct((_B, 1, _M), f32),
        ],
        interpret=_interp,
    )(S, Level[:, None, :])

    VR = Vflat.reshape(_B, _M, _HW)

    G, rowsum = pl.pallas_call(
        _histb_body,
        grid=(_B, _NPH),
        in_specs=[pl.BlockSpec((1, _M, _PBH), lambda b, p: (b, 0, p))],
        out_specs=[
            pl.BlockSpec((1, _M, _M), lambda b, p: (b, 0, 0)),
            pl.BlockSpec((1, 1, _M), lambda b, p: (b, 0, 0)),
        ],
        out_shape=[
            jax.ShapeDtypeStruct((_B, _M, _M), f32),
            jax.ShapeDtypeStruct((_B, 1, _M), f32),
        ],
        interpret=_interp,
    )(VR)

    Vsum2 = Vsum[:, 0, :]
    Vtot = jnp.sum(Vsum2, axis=1)
    Chist2 = jnp.stack([Vsum2 / Vtot[:, None], Level], axis=-1).reshape(-1, 2)
    fcout = pl.pallas_call(
        _fc_body,
        in_specs=[
            pl.BlockSpec((_B * _M, 2), lambda: (0, 0)),
            pl.BlockSpec((2, _C), lambda: (0, 0)),
            pl.BlockSpec((1, _C), lambda: (0, 0)),
        ],
        out_specs=pl.BlockSpec((_B * _M, _C), lambda: (0, 0)),
        out_shape=jax.ShapeDtypeStruct((_B * _M, _C), f32),
        interpret=_interp,
    )(Chist2, Wfc.T, bfc.reshape(1, _C))
    Ch = fcout.reshape(_B, _C, _M)

    Lq, b1 = pl.pallas_call(
        _small_body,
        in_specs=[
            pl.BlockSpec((_B, _C, _M), lambda: (0, 0, 0)),
            pl.BlockSpec((_C, _C), lambda: (0, 0)),
            pl.BlockSpec((_C, _C), lambda: (0, 0)),
            pl.BlockSpec((_C, _C), lambda: (0, 0)),
            pl.BlockSpec((3, _C), lambda: (0, 0)),
            pl.BlockSpec((_B, _M), lambda: (0, 0)),
            pl.BlockSpec((_B, _M, _M), lambda: (0, 0, 0)),
            pl.BlockSpec((2, _C), lambda: (0, 0)),
        ],
        out_specs=[
            pl.BlockSpec((_B, _C, _M), lambda: (0, 0, 0)),
            pl.BlockSpec((1, _C), lambda: (0, 0)),
        ],
        out_shape=[
            jax.ShapeDtypeStruct((_B, _C, _M), f32),
            jax.ShapeDtypeStruct((1, _C), f32),
        ],
        interpret=_interp,
    )(Ch, Wp1, Wp2, Wp3, jnp.stack([bp1, bp2, bp3]), rowsum[:, 0, :], G,
      jnp.stack([bn1_g, bn1_b]))

    w2 = W2[:, 0, 0, 0]
    sS = jnp.sum(ssum[:, 0, 0])
    ssS = jnp.sum(ssq[:, 0, 0])
    mS = sS / _N
    vS = ssS / _N - mS * mS
    m2 = w2 * mS
    v2 = w2 * w2 * vS
    a2 = bn2_g / jnp.sqrt(v2 + _EPS)
    cpack = jnp.stack([coefE[0], coefE[1], a2 * w2, bn2_b - m2 * a2, b1[0],
                       jnp.zeros_like(w2)])

    out = pl.pallas_call(
        _p6_body,
        grid=(_B, _NP),
        in_specs=[
            pl.BlockSpec((1, _C, _PB), lambda b, p: (b, 0, p)),
            pl.BlockSpec((1, _C, _PB), lambda b, p: (b, 0, p)),
            pl.BlockSpec((1, 1, _PB), lambda b, p: (b, 0, p)),
            pl.BlockSpec((1, _M, _PB), lambda b, p: (b, 0, p)),
            pl.BlockSpec((1, _C, _M), lambda b, p: (b, 0, 0)),
            pl.BlockSpec((6, _C), lambda b, p: (0, 0)),
        ],
        out_specs=pl.BlockSpec((1, _C, _PB), lambda b, p: (b, 0, p)),
        out_shape=jax.ShapeDtypeStruct((_B, _C, _HW), f32),
        interpret=_interp,
    )(xf, h4.reshape(_B, _C, _HW), S, VR, Lq, cpack)

    return out.reshape(_B, _C, _H, _W)


# speedup vs baseline: 2.0732x; 1.0007x over previous
import functools
import jax
import jax.numpy as jnp
from jax import lax
from jax.experimental import pallas as pl

_M = 10
_B, _C, _H, _W = 4, 96, 224, 224
_HW = _H * _W
_N = _B * _HW
_PB = 1792
_NP = _HW // _PB
_PBH = 6272
_NPH = _HW // _PBH
_CB = 8
_NC = _C // _CB
_EPS = 1e-5

_BF = jnp.bfloat16


def _p1_body(x_ref, w1_ref, xc1_ref, nb2_ref, statA_ref, xg_ref):
    b, p = pl.program_id(0), pl.program_id(1)
    del b
    xb = x_ref[0]
    y = lax.dot_general(w1_ref[...], xb, (((1,), (0,)), ((), ())),
                        preferred_element_type=jnp.float32)
    xc1_ref[0] = y.astype(_BF)
    nb2_ref[0, 0] = jnp.sum(xb * xb, axis=0)
    st = jnp.stack([jnp.sum(y, axis=1), jnp.sum(y * y, axis=1)])

    @pl.when(p == 0)
    def _():
        statA_ref[...] = jnp.zeros_like(statA_ref)
        xg_ref[...] = jnp.zeros_like(xg_ref)

    statA_ref[0] += st
    xg_ref[...] += jnp.sum(xb, axis=1).reshape(1, 1, _C)


def _phs_body(x_ref, xg_ref, nb2_ref, s_ref, smin_ref, smax_ref, ssum_ref,
              ssq_ref):
    p = pl.program_id(1)
    xb = x_ref[0]
    xg = xg_ref[0, 0]
    na = jnp.sqrt(jnp.sum(xg * xg))
    dot = jnp.sum(xg[:, None] * xb, axis=0)
    nb = jnp.sqrt(nb2_ref[0, 0])
    e8 = 1e-8
    s = dot / (jnp.maximum(na, e8) * jnp.maximum(nb, e8))
    s_ref[0, 0] = s

    @pl.when(p == 0)
    def _():
        smin_ref[...] = jnp.full((1, 1, 1), jnp.inf, jnp.float32)
        smax_ref[...] = jnp.full((1, 1, 1), -jnp.inf, jnp.float32)
        ssum_ref[...] = jnp.zeros((1, 1, 1), jnp.float32)
        ssq_ref[...] = jnp.zeros((1, 1, 1), jnp.float32)

    smin_ref[...] = jnp.minimum(smin_ref[...], jnp.min(s).reshape(1, 1, 1))
    smax_ref[...] = jnp.maximum(smax_ref[...], jnp.max(s).reshape(1, 1, 1))
    ssum_ref[...] += jnp.sum(s).reshape(1, 1, 1)
    ssq_ref[...] += jnp.sum(s * s).reshape(1, 1, 1)


def _conv_body(horiz, dil, has_bias, a_ref, coef_ref, w_ref, bias_ref,
               h_ref, stat_ref):
    b = pl.program_id(1)
    sc = coef_ref[0, 0, :].astype(_BF)
    of = coef_ref[0, 1, :].astype(_BF)
    wb = w_ref[...].astype(_BF)
    a = a_ref[0] * sc[:, None, None] + of[:, None, None]
    pad = 2 * dil
    if horiz:
        z = jnp.zeros((_CB, _H, pad), _BF)
        ap = jnp.concatenate([z, a, z], axis=2)
        h = sum(wb[:, t][:, None, None] * ap[:, :, t * dil:t * dil + _W]
                for t in range(5))
    else:
        z = jnp.zeros((_CB, pad, _W), _BF)
        ap = jnp.concatenate([z, a, z], axis=1)
        h = sum(wb[:, t][:, None, None] * ap[:, t * dil:t * dil + _H, :]
                for t in range(5))
    if has_bias:
        h = h + bias_ref[0, 0, :].astype(_BF)[:, None, None]
    h_ref[0] = h
    hf = h.astype(jnp.float32)
    st = jnp.stack([jnp.sum(hf, axis=(1, 2)), jnp.sum(hf * hf, axis=(1, 2))])

    @pl.when(b == 0)
    def _():
        stat_ref[...] = jnp.zeros_like(stat_ref)

    stat_ref[0] += st


def _hista_body(s_ref, lev_ref, v_ref, vsum_ref):
    p = pl.program_id(1)
    s = s_ref[0, 0]
    L = lev_ref[0, 0]
    diff = jnp.abs(L[None, :] - s[:, None])
    v = jnp.where(diff < 0.5 / _M, 1.0 - diff, 0.0)
    v_ref[0] = v.astype(_BF)

    @pl.when(p == 0)
    def _():
        vsum_ref[...] = jnp.zeros_like(vsum_ref)

    vsum_ref[0] += jnp.sum(v, axis=0).reshape(1, _M)


def _histb_body(vr_ref, g_ref, rs_ref):
    p = pl.program_id(1)
    u = vr_ref[0]
    g = lax.dot_general(u, u, (((1,), (1,)), ((), ())),
                        preferred_element_type=jnp.float32)
    uf = u.astype(jnp.float32)

    @pl.when(p == 0)
    def _():
        g_ref[...] = jnp.zeros_like(g_ref)
        rs_ref[...] = jnp.zeros_like(rs_ref)

    g_ref[0] += g
    rs_ref[0] += jnp.sum(uf, axis=1).reshape(1, _M)


def _fc_body(ch_ref, wfc_ref, bfc_ref, o_ref):
    c0 = ch_ref[:, 0][:, None]
    c1 = ch_ref[:, 1][:, None]
    o_ref[...] = (c0 * wfc_ref[0, :][None, :] + c1 * wfc_ref[1, :][None, :]
                  + bfc_ref[0, :][None, :])


def _small_body(ch_ref, wp1_ref, wp2_ref, wp3_ref, bp_ref, rs_ref, g_ref,
                bn1_ref, lq_ref, b1_ref):
    hp = lax.Precision.HIGHEST
    s1 = jnp.zeros((_C,), jnp.float32)
    ss1 = jnp.zeros((_C,), jnp.float32)
    lps = []
    for b in range(_B):
        chb = ch_ref[b]
        p1 = lax.dot_general(wp1_ref[...], chb, (((1,), (0,)), ((), ())),
                             preferred_element_type=jnp.float32, precision=hp) \
            + bp_ref[0, :][:, None]
        p2 = lax.dot_general(wp2_ref[...], chb, (((1,), (0,)), ((), ())),
                             preferred_element_type=jnp.float32, precision=hp) \
            + bp_ref[1, :][:, None]
        p3 = lax.dot_general(wp3_ref[...], chb, (((1,), (0,)), ((), ())),
                             preferred_element_type=jnp.float32, precision=hp) \
            + bp_ref[2, :][:, None]
        logits = lax.dot_general(p1, p2, (((0,), (0,)), ((), ())),
                                 preferred_element_type=jnp.float32,
                                 precision=hp)
        e = jnp.exp(logits - jnp.max(logits, axis=-1, keepdims=True))
        xm = e / jnp.sum(e, axis=-1, keepdims=True)
        lp = lax.dot_general(p3, xm, (((1,), (0,)), ((), ())),
                             preferred_element_type=jnp.float32, precision=hp)
        lps.append(lp)
        s1 = s1 + jnp.sum(lp * rs_ref[b][None, :], axis=1)
        lg = lax.dot_general(lp, g_ref[b], (((1,), (0,)), ((), ())),
                             preferred_element_type=jnp.float32, precision=hp)
        ss1 = ss1 + jnp.sum(lg * lp, axis=1)
    m1 = s1 / _N
    v1 = ss1 / _N - m1 * m1
    a1 = bn1_ref[0, :] / jnp.sqrt(v1 + _EPS)
    b1_ref[...] = (bn1_ref[1, :] - m1 * a1).reshape(1, _C)
    for b in range(_B):
        lq_ref[b] = lps[b] * a1[:, None]


def _p6_body(x_ref, h4_ref, s_ref, vr_ref, lq_ref, cp_ref, o_ref):
    aE = cp_ref[0, :][:, None]
    bE = cp_ref[1, :][:, None]
    a2w = cp_ref[2, :][:, None]
    b2 = cp_ref[3, :][:, None]
    b1 = cp_ref[4, :][:, None]
    xc = h4_ref[0].astype(jnp.float32) * aE + bE
    tex = lax.dot_general(lq_ref[0].astype(_BF), vr_ref[0],
                          (((1,), (0,)), ((), ())),
                          preferred_element_type=jnp.float32) + b1
    s = s_ref[0, 0][None, :]
    tf = a2w * s + b2
    o_ref[0] = x_ref[0] * jax.nn.sigmoid(tf + tex + xc)


def _coef(stat, g, b):
    m = stat[0] / _N
    v = stat[1] / _N - m * m
    a = g / jnp.sqrt(v + _EPS)
    return jnp.stack([a, b - m * a])


def kernel(x, W1, bnA_g, bnA_b, prw1, bnB_g, bnB_b, prw2, bnC_g, bnC_b, prw3,
           prb3, bnD_g, bnD_b, prw4, prb4, bnE_g, bnE_b, Wfc, bfc, Wp1, bp1,
           Wp2, bp2, Wp3, bp3, W2, bn1_g, bn1_b, bn2_g, bn2_b):
    f32 = jnp.float32
    xf = x.reshape(_B, _C, _HW)

    xc1, nb2, statA, xgsum = pl.pallas_call(
        _p1_body,
        grid=(_B, _NP),
        in_specs=[
            pl.BlockSpec((1, _C, _PB), lambda b, p: (b, 0, p)),
            pl.BlockSpec((_C, _C), lambda b, p: (0, 0)),
        ],
        out_specs=[
            pl.BlockSpec((1, _C, _PB), lambda b, p: (b, 0, p)),
            pl.BlockSpec((1, 1, _PB), lambda b, p: (b, 0, p)),
            pl.BlockSpec((1, 2, _C), lambda b, p: (0, 0, 0)),
            pl.BlockSpec((1, 1, _C), lambda b, p: (b, 0, 0)),
        ],
        out_shape=[
            jax.ShapeDtypeStruct((_B, _C, _HW), _BF),
            jax.ShapeDtypeStruct((_B, 1, _HW), f32),
            jax.ShapeDtypeStruct((1, 2, _C), f32),
            jax.ShapeDtypeStruct((_B, 1, _C), f32),
        ],
    )(xf, W1[:, :, 0, 0])
    xg = xgsum * (1.0 / _HW)

    S, smin, smax, ssum, ssq = pl.pallas_call(
        _phs_body,
        grid=(_B, _NP),
        in_specs=[
            pl.BlockSpec((1, _C, _PB), lambda b, p: (b, 0, p)),
            pl.BlockSpec((1, 1, _C), lambda b, p: (b, 0, 0)),
            pl.BlockSpec((1, 1, _PB), lambda b, p: (b, 0, p)),
        ],
        out_specs=[
            pl.BlockSpec((1, 1, _PB), lambda b, p: (b, 0, p)),
            pl.BlockSpec((1, 1, 1), lambda b, p: (b, 0, 0)),
            pl.BlockSpec((1, 1, 1), lambda b, p: (b, 0, 0)),
            pl.BlockSpec((1, 1, 1), lambda b, p: (b, 0, 0)),
            pl.BlockSpec((1, 1, 1), lambda b, p: (b, 0, 0)),
        ],
        out_shape=[
            jax.ShapeDtypeStruct((_B, 1, _HW), f32),
            jax.ShapeDtypeStruct((_B, 1, 1), f32),
            jax.ShapeDtypeStruct((_B, 1, 1), f32),
            jax.ShapeDtypeStruct((_B, 1, 1), f32),
            jax.ShapeDtypeStruct((_B, 1, 1), f32),
        ],
    )(xf, xg, nb2)

    def conv_pass(a, coefs, w, bias, horiz, dil):
        body = functools.partial(_conv_body, horiz, dil, bias is not None)
        cf = coefs.reshape(2, _NC, _CB).transpose(1, 0, 2)
        bs = (bias if bias is not None else jnp.zeros((_C,), f32))
        bs = bs.reshape(_NC, 1, _CB)
        h, stat = pl.pallas_call(
            body,
            grid=(_NC, _B),
            in_specs=[
                pl.BlockSpec((1, _CB, _H, _W), lambda c, b: (b, c, 0, 0)),
                pl.BlockSpec((1, 2, _CB), lambda c, b: (c, 0, 0)),
                pl.BlockSpec((_CB, 5), lambda c, b: (c, 0)),
                pl.BlockSpec((1, 1, _CB), lambda c, b: (c, 0, 0)),
            ],
            out_specs=[
                pl.BlockSpec((1, _CB, _H, _W), lambda c, b: (b, c, 0, 0)),
                pl.BlockSpec((1, 2, _CB), lambda c, b: (c, 0, 0)),
            ],
            out_shape=[
                jax.ShapeDtypeStruct((_B, _C, _H, _W), _BF),
                jax.ShapeDtypeStruct((_NC, 2, _CB), f32),
            ],
            )(a.reshape(_B, _C, _H, _W), cf, w, bs)
        return h, stat.transpose(1, 0, 2).reshape(2, _C)

    coefA = _coef(statA[0], bnA_g, bnA_b)
    h1, statB = conv_pass(xc1, coefA, prw1[:, 0, 0, :], None, True, 1)
    coefB = _coef(statB, bnB_g, bnB_b)
    h2, statC = conv_pass(h1, coefB, prw2[:, 0, :, 0], None, False, 1)
    coefC = _coef(statC, bnC_g, bnC_b)
    h3, statD = conv_pass(h2, coefC, prw3[:, 0, 0, :], prb3, True, 2)
    coefD = _coef(statD, bnD_g, bnD_b)
    h4, statE = conv_pass(h3, coefD, prw4[:, 0, :, 0], prb4, False, 2)
    coefE = _coef(statE, bnE_g, bnE_b)

    mn = smin[:, 0, 0]
    mx = smax[:, 0, 0]
    t = jnp.linspace(0.0, 1.0, _M).astype(f32)
    Level = mn[:, None] + (mx - mn)[:, None] * t[None, :]

    Vflat, Vsum = pl.pallas_call(
        _hista_body,
        grid=(_B, _NPH),
        in_specs=[
            pl.BlockSpec((1, 1, _PBH), lambda b, p: (b, 0, p)),
            pl.BlockSpec((1, 1, _M), lambda b, p: (b, 0, 0)),
        ],
        out_specs=[
            pl.BlockSpec((1, _PBH, _M), lambda b, p: (b, p, 0)),
            pl.BlockSpec((1, 1, _M), lambda b, p: (b, 0, 0)),
        ],
        out_shape=[
            jax.ShapeDtypeStruct((_B, _HW, _M), _BF),
            jax.ShapeDtypeStruct((_B, 1, _M), f32),
        ],
    )(S, Level[:, None, :])

    VR = Vflat.reshape(_B, _M, _HW)

    G, rowsum = pl.pallas_call(
        _histb_body,
        grid=(_B, _NPH),
        in_specs=[pl.BlockSpec((1, _M, _PBH), lambda b, p: (b, 0, p))],
        out_specs=[
            pl.BlockSpec((1, _M, _M), lambda b, p: (b, 0, 0)),
            pl.BlockSpec((1, 1, _M), lambda b, p: (b, 0, 0)),
        ],
        out_shape=[
            jax.ShapeDtypeStruct((_B, _M, _M), f32),
            jax.ShapeDtypeStruct((_B, 1, _M), f32),
        ],
    )(VR)

    Vsum2 = Vsum[:, 0, :]
    Vtot = jnp.sum(Vsum2, axis=1)
    Chist2 = jnp.stack([Vsum2 / Vtot[:, None], Level], axis=-1).reshape(-1, 2)
    fcout = pl.pallas_call(
        _fc_body,
        in_specs=[
            pl.BlockSpec((_B * _M, 2), lambda: (0, 0)),
            pl.BlockSpec((2, _C), lambda: (0, 0)),
            pl.BlockSpec((1, _C), lambda: (0, 0)),
        ],
        out_specs=pl.BlockSpec((_B * _M, _C), lambda: (0, 0)),
        out_shape=jax.ShapeDtypeStruct((_B * _M, _C), f32),
    )(Chist2, Wfc.T, bfc.reshape(1, _C))
    Ch = fcout.reshape(_B, _C, _M)

    Lq, b1 = pl.pallas_call(
        _small_body,
        in_specs=[
            pl.BlockSpec((_B, _C, _M), lambda: (0, 0, 0)),
            pl.BlockSpec((_C, _C), lambda: (0, 0)),
            pl.BlockSpec((_C, _C), lambda: (0, 0)),
            pl.BlockSpec((_C, _C), lambda: (0, 0)),
            pl.BlockSpec((3, _C), lambda: (0, 0)),
            pl.BlockSpec((_B, _M), lambda: (0, 0)),
            pl.BlockSpec((_B, _M, _M), lambda: (0, 0, 0)),
            pl.BlockSpec((2, _C), lambda: (0, 0)),
        ],
        out_specs=[
            pl.BlockSpec((_B, _C, _M), lambda: (0, 0, 0)),
            pl.BlockSpec((1, _C), lambda: (0, 0)),
        ],
        out_shape=[
            jax.ShapeDtypeStruct((_B, _C, _M), f32),
            jax.ShapeDtypeStruct((1, _C), f32),
        ],
    )(Ch, Wp1, Wp2, Wp3, jnp.stack([bp1, bp2, bp3]), rowsum[:, 0, :], G,
      jnp.stack([bn1_g, bn1_b]))

    w2 = W2[:, 0, 0, 0]
    sS = jnp.sum(ssum[:, 0, 0])
    ssS = jnp.sum(ssq[:, 0, 0])
    mS = sS / _N
    vS = ssS / _N - mS * mS
    m2 = w2 * mS
    v2 = w2 * w2 * vS
    a2 = bn2_g / jnp.sqrt(v2 + _EPS)
    cpack = jnp.stack([coefE[0], coefE[1], a2 * w2, bn2_b - m2 * a2, b1[0],
                       jnp.zeros_like(w2)])

    out = pl.pallas_call(
        _p6_body,
        grid=(_B, _NP),
        in_specs=[
            pl.BlockSpec((1, _C, _PB), lambda b, p: (b, 0, p)),
            pl.BlockSpec((1, _C, _PB), lambda b, p: (b, 0, p)),
            pl.BlockSpec((1, 1, _PB), lambda b, p: (b, 0, p)),
            pl.BlockSpec((1, _M, _PB), lambda b, p: (b, 0, p)),
            pl.BlockSpec((1, _C, _M), lambda b, p: (b, 0, 0)),
            pl.BlockSpec((6, _C), lambda b, p: (0, 0)),
        ],
        out_specs=pl.BlockSpec((1, _C, _PB), lambda b, p: (b, 0, p)),
        out_shape=jax.ShapeDtypeStruct((_B, _C, _HW), f32),
    )(xf, h4.reshape(_B, _C, _HW), S, VR, Lq, cpack)

    return out.reshape(_B, _C, _H, _W)
